# Initial kernel scaffold; baseline (speedup 1.0000x reference)
import functools

import jax
import jax.numpy as jnp
from jax import lax
from jax.experimental import pallas as pl
from jax.experimental.pallas import tpu as pltpu
from jax.experimental.pallas import tpu_sc as plsc

N_NODES = 10000
N_EDGES = 320000
D = 128
ALPHA = 0.5

NC, NS = 2, 16
NPAD = 10240
ROWS_PER_SUB = NPAD // NS
CHUNK_ROWS = 4
N_CHUNKS = N_EDGES // (CHUNK_ROWS * 128)
IDX_ROWS = N_EDGES // 128


def _sc_mesh():
    return plsc.VectorSubcoreMesh(core_axis_name="c", subcore_axis_name="s")


def _degree_kernel(eidx_r, zvec):
    @functools.partial(
        pl.kernel,
        out_type=jax.ShapeDtypeStruct((NC, NPAD), jnp.float32),
        mesh=_sc_mesh(),
        scratch_types=[
            pltpu.VMEM((128,), jnp.int32),
            pltpu.VMEM((128,), jnp.float32),
            pltpu.VMEM_SHARED((NPAD,), jnp.float32),
        ],
    )
    def k(eidx_hbm, zvec_hbm, deg_hbm, idx_v, ones_v, acc):
        c = lax.axis_index("c")
        s = lax.axis_index("s")
        pltpu.sync_copy(zvec_hbm, acc.at[pl.ds(s * ROWS_PER_SUB, ROWS_PER_SUB)])

        @pl.loop(0, 128, step=16)
        def _(i):
            ones_v[pl.ds(i, 16)] = jnp.full((16,), 1.0, jnp.float32)

        plsc.subcore_barrier()

        @pl.loop(s, IDX_ROWS, step=NS)
        def _(chunk):
            pltpu.sync_copy(eidx_hbm.at[c, chunk], idx_v)
            pltpu.sync_copy(ones_v, acc.at[idx_v], add=True)

        plsc.subcore_barrier()
        sl = pl.ds(s * ROWS_PER_SUB, ROWS_PER_SUB)
        pltpu.sync_copy(acc.at[sl], deg_hbm.at[c, sl])

    return k(eidx_r, zvec)


def _aggregate_kernel(zcat, gidx_r, sidx_r, zrows):
    @functools.partial(
        pl.kernel,
        out_type=jax.ShapeDtypeStruct((NC, N_NODES, D), jnp.float32),
        mesh=_sc_mesh(),
        scratch_types=[
            pltpu.VMEM((CHUNK_ROWS, 128), jnp.int32),
            pltpu.VMEM((CHUNK_ROWS, 128), jnp.int32),
            pltpu.VMEM((CHUNK_ROWS * 128, D), jnp.float32),
            pltpu.VMEM_SHARED((NPAD, D), jnp.float32),
        ],
    )
    def k(zcat_hbm, gidx_hbm, sidx_hbm, zrows_hbm, s_hbm, gi_v, si_v, gbuf, acc):
        c = lax.axis_index("c")
        s = lax.axis_index("s")
        pltpu.sync_copy(zrows_hbm, acc.at[pl.ds(s * ROWS_PER_SUB, ROWS_PER_SUB)])
        plsc.subcore_barrier()

        @pl.loop(s, N_CHUNKS, step=NS)
        def _(chunk):
            row0 = chunk * CHUNK_ROWS
            pltpu.sync_copy(gidx_hbm.at[c, pl.ds(row0, CHUNK_ROWS)], gi_v)
            pltpu.sync_copy(sidx_hbm.at[c, pl.ds(row0, CHUNK_ROWS)], si_v)
            for j in range(CHUNK_ROWS):
                pltpu.sync_copy(zcat_hbm.at[gi_v.at[j]],
                                gbuf.at[pl.ds(j * 128, 128)])
            for j in range(CHUNK_ROWS):
                pltpu.sync_copy(gbuf.at[pl.ds(j * 128, 128)],
                                acc.at[si_v.at[j]], add=True)

        plsc.subcore_barrier()
        rows_out = N_NODES // NS
        sl = pl.ds(s * rows_out, rows_out)
        pltpu.sync_copy(acc.at[sl], s_hbm.at[c, sl])

    return k(zcat, gidx_r, sidx_r, zrows)


_BLK = 1000
_NBLK = N_NODES // _BLK


def _transform_body(x_ref, w_ref, deg_ref, z_ref):
    i = pl.program_id(0)
    y = jnp.dot(x_ref[...], w_ref[...], precision=lax.Precision.HIGHEST,
                preferred_element_type=jnp.float32)
    dseg = deg_ref[:, pl.ds(i * _BLK, _BLK)]
    oinv = jnp.where(dseg[0] > 0, lax.rsqrt(dseg[0]), 0.0)
    iinv = jnp.where(dseg[1] > 0, lax.rsqrt(dseg[1]), 0.0)
    z_ref[0] = iinv[:, None] * y[:, :D]
    z_ref[1] = oinv[:, None] * y[:, D:]


def _transform(x, wcat, deg):
    return pl.pallas_call(
        _transform_body,
        grid=(_NBLK,),
        in_specs=[
            pl.BlockSpec((_BLK, D), lambda i: (i, 0)),
            pl.BlockSpec((D, 2 * D), lambda i: (0, 0)),
            pl.BlockSpec((2, N_NODES), lambda i: (0, 0)),
        ],
        out_specs=pl.BlockSpec((2, _BLK, D), lambda i: (0, i, 0)),
        out_shape=jax.ShapeDtypeStruct((2, N_NODES, D), jnp.float32),
    )(x, wcat, deg)


def _combine_body(s_ref, deg_ref, b_ref, o_ref):
    i = pl.program_id(0)
    dseg = deg_ref[:, pl.ds(i * _BLK, _BLK)]
    oinv = jnp.where(dseg[0] > 0, lax.rsqrt(dseg[0]), 0.0)
    iinv = jnp.where(dseg[1] > 0, lax.rsqrt(dseg[1]), 0.0)
    o_ref[...] = (oinv[:, None] * s_ref[0] + iinv[:, None] * s_ref[1]
                  + b_ref[...])


def _combine(S, deg, bias):
    return pl.pallas_call(
        _combine_body,
        grid=(_NBLK,),
        in_specs=[
            pl.BlockSpec((2, _BLK, D), lambda i: (0, i, 0)),
            pl.BlockSpec((2, N_NODES), lambda i: (0, 0)),
            pl.BlockSpec((1, D), lambda i: (0, 0)),
        ],
        out_specs=pl.BlockSpec((_BLK, D), lambda i: (i, 0)),
        out_shape=jax.ShapeDtypeStruct((N_NODES, D), jnp.float32),
    )(S, deg, bias)


def kernel(x, edge_index, W_src, b_src, W_dst, b_dst):
    eidx_r = edge_index.reshape(NC, IDX_ROWS, 128)
    zvec = jnp.zeros((ROWS_PER_SUB,), jnp.float32)
    zrows = jnp.zeros((ROWS_PER_SUB, D), jnp.float32)

    deg = _degree_kernel(eidx_r, zvec)[:, :N_NODES]

    wcat = jnp.concatenate([ALPHA * W_src, (1.0 - ALPHA) * W_dst], axis=1)
    zcat = _transform(x, wcat, deg).reshape(2 * N_NODES, D)

    gidx = edge_index[::-1] + jnp.array([[0], [N_NODES]], jnp.int32)
    gidx_r = gidx.reshape(NC, IDX_ROWS, 128)
    S = _aggregate_kernel(zcat, gidx_r, eidx_r, zrows)

    bias = (ALPHA * b_src + (1.0 - ALPHA) * b_dst).reshape(1, D)
    return _combine(S, deg, bias)


# speedup vs baseline: 15.3442x; 15.3442x over previous
import functools

import jax
import jax.numpy as jnp
from jax import lax
from jax.experimental import pallas as pl
from jax.experimental.pallas import tpu as pltpu
from jax.experimental.pallas import tpu_sc as plsc

N_NODES = 10000
N_EDGES = 320000
D = 128
ALPHA = 0.5

NC, NS = 2, 16
NPAD = 10240
ROWS_PER_SUB = NPAD // NS
CHUNK_ROWS = 2
N_CHUNKS = N_EDGES // (CHUNK_ROWS * 128)
IDX_ROWS = N_EDGES // 128


def _sc_mesh():
    return plsc.VectorSubcoreMesh(core_axis_name="c", subcore_axis_name="s")


def _degree_kernel(eidx_r, zvec):
    @functools.partial(
        pl.kernel,
        out_type=jax.ShapeDtypeStruct((NC, NPAD), jnp.float32),
        mesh=_sc_mesh(),
        scratch_types=[
            pltpu.VMEM((128,), jnp.int32),
            pltpu.VMEM((128,), jnp.float32),
            pltpu.VMEM_SHARED((NPAD,), jnp.float32),
        ],
    )
    def k(eidx_hbm, zvec_hbm, deg_hbm, idx_v, ones_v, acc):
        c = lax.axis_index("c")
        s = lax.axis_index("s")
        pltpu.sync_copy(zvec_hbm, acc.at[pl.ds(s * ROWS_PER_SUB, ROWS_PER_SUB)])

        @pl.loop(0, 128, step=16)
        def _(i):
            ones_v[pl.ds(i, 16)] = jnp.full((16,), 1.0, jnp.float32)

        plsc.subcore_barrier()

        @pl.loop(s, IDX_ROWS, step=NS)
        def _(chunk):
            pltpu.sync_copy(eidx_hbm.at[c, chunk], idx_v)
            pltpu.sync_copy(ones_v, acc.at[idx_v], add=True)

        plsc.subcore_barrier()
        sl = pl.ds(s * ROWS_PER_SUB, ROWS_PER_SUB)
        pltpu.sync_copy(acc.at[sl], deg_hbm.at[c, sl])

    return k(eidx_r, zvec)


def _aggregate_kernel(zcat, gidx_r, sidx_r, zrows):
    @functools.partial(
        pl.kernel,
        out_type=jax.ShapeDtypeStruct((NC, NPAD, D), jnp.float32),
        mesh=_sc_mesh(),
        scratch_types=[
            pltpu.VMEM((CHUNK_ROWS, 128), jnp.int32),
            pltpu.VMEM((CHUNK_ROWS, 128), jnp.int32),
            pltpu.VMEM((CHUNK_ROWS * 128, D), jnp.float32),
            pltpu.VMEM_SHARED((NPAD, D), jnp.float32),
        ],
    )
    def k(zcat_hbm, gidx_hbm, sidx_hbm, zrows_hbm, s_hbm, gi_v, si_v, gbuf, acc):
        c = lax.axis_index("c")
        s = lax.axis_index("s")
        pltpu.sync_copy(zrows_hbm, acc.at[pl.ds(s * ROWS_PER_SUB, ROWS_PER_SUB)])
        plsc.subcore_barrier()

        @pl.loop(s, N_CHUNKS, step=NS)
        def _(chunk):
            row0 = chunk * CHUNK_ROWS
            pltpu.sync_copy(gidx_hbm.at[c, pl.ds(row0, CHUNK_ROWS)], gi_v)
            pltpu.sync_copy(sidx_hbm.at[c, pl.ds(row0, CHUNK_ROWS)], si_v)
            for j in range(CHUNK_ROWS):
                pltpu.sync_copy(zcat_hbm.at[gi_v.at[j]],
                                gbuf.at[pl.ds(j * 128, 128)])
            for j in range(CHUNK_ROWS):
                pltpu.sync_copy(gbuf.at[pl.ds(j * 128, 128)],
                                acc.at[si_v.at[j]], add=True)

        plsc.subcore_barrier()
        sl = pl.ds(s * ROWS_PER_SUB, ROWS_PER_SUB)
        pltpu.sync_copy(acc.at[sl], s_hbm.at[c, sl])

    return k(zcat, gidx_r, sidx_r, zrows)


_BLK = 1000
_NBLK = N_NODES // _BLK


def _inv_sqrt_cols(degt):
    oinv = jnp.where(degt[:, 0:1] > 0, lax.rsqrt(degt[:, 0:1]), 0.0)
    iinv = jnp.where(degt[:, 1:2] > 0, lax.rsqrt(degt[:, 1:2]), 0.0)
    return oinv, iinv


def _transform_body(x_ref, w_ref, degt_ref, z_ref):
    y = jnp.dot(x_ref[...], w_ref[...], precision=lax.Precision.HIGHEST,
                preferred_element_type=jnp.float32)
    oinv, iinv = _inv_sqrt_cols(degt_ref[...])
    z_ref[0] = iinv * y[:, :D]
    z_ref[1] = oinv * y[:, D:]


def _transform(x, wcat, degt):
    return pl.pallas_call(
        _transform_body,
        grid=(_NBLK,),
        in_specs=[
            pl.BlockSpec((_BLK, D), lambda i: (i, 0)),
            pl.BlockSpec((D, 2 * D), lambda i: (0, 0)),
            pl.BlockSpec((_BLK, 2), lambda i: (i, 0)),
        ],
        out_specs=pl.BlockSpec((2, _BLK, D), lambda i: (0, i, 0)),
        out_shape=jax.ShapeDtypeStruct((2, N_NODES, D), jnp.float32),
    )(x, wcat, degt)


def _combine_body(s_ref, degt_ref, b_ref, o_ref):
    oinv, iinv = _inv_sqrt_cols(degt_ref[...])
    o_ref[...] = oinv * s_ref[0] + iinv * s_ref[1] + b_ref[...]


def _combine(S, degt, bias):
    return pl.pallas_call(
        _combine_body,
        grid=(_NBLK,),
        in_specs=[
            pl.BlockSpec((2, _BLK, D), lambda i: (0, i, 0)),
            pl.BlockSpec((_BLK, 2), lambda i: (i, 0)),
            pl.BlockSpec((1, D), lambda i: (0, 0)),
        ],
        out_specs=pl.BlockSpec((_BLK, D), lambda i: (i, 0)),
        out_shape=jax.ShapeDtypeStruct((N_NODES, D), jnp.float32),
    )(S, degt, bias)


def kernel(x, edge_index, W_src, b_src, W_dst, b_dst):
    eidx_r = edge_index.reshape(NC, IDX_ROWS, 128)
    zvec = jnp.zeros((ROWS_PER_SUB,), jnp.float32)
    zrows = jnp.zeros((ROWS_PER_SUB, D), jnp.float32)

    degt = _degree_kernel(eidx_r, zvec)[:, :N_NODES].T

    wcat = jnp.concatenate([ALPHA * W_src, (1.0 - ALPHA) * W_dst], axis=1)
    zcat = _transform(x, wcat, degt).reshape(2 * N_NODES, D)

    gidx = edge_index[::-1] + jnp.array([[0], [N_NODES]], jnp.int32)
    gidx_r = gidx.reshape(NC, IDX_ROWS, 128)
    S = _aggregate_kernel(zcat, gidx_r, eidx_r, zrows)

    bias = (ALPHA * b_src + (1.0 - ALPHA) * b_dst).reshape(1, D)
    return _combine(S, degt, bias)
